# baseline (speedup 1.0000x reference)
import functools

import jax
import jax.numpy as jnp
from jax import lax
from jax.experimental import pallas as pl
from jax.experimental.pallas import tpu as pltpu
from jax.experimental.pallas import tpu_sc as plsc

B = 4096
L = 50
F = 13
V = 100000
D = 16
C = 16

NC = 2
NS = 16
NW = NC * NS
BPW = B // NW
FA = 7


def _zero_racc(racc_v):
    def zero16(i, _):
        racc_v[i] = jnp.zeros((D,), jnp.float32)
        return 0

    lax.fori_loop(0, BPW, zero16, 0)


def _accumulate(racc_v, rows_v, wf):
    def acc8(i, _):
        for j in range(8):
            plsc.addupdate(racc_v.at[i * 8 + j], rows_v[i * 8 + j] * wf)
        return 0

    lax.fori_loop(0, BPW // 8, acc8, 0)


def _flat_gather_pass(idx_src, table, woff, b0, w_v, idx_v, rows_v, racc_v,
                      sem):
    def fbody(f, _):
        pltpu.sync_copy(idx_src.at[pl.ds(f * B + b0, BPW)], idx_v)

        def add16(k, _):
            s = pl.ds(k * 16, 16)
            idx_v[s] = idx_v[s] + f * V
            return 0

        lax.fori_loop(0, BPW // 16, add16, 0)
        pltpu.async_copy(table.at[idx_v], rows_v, sem).wait()
        wf = w_v[pl.ds(woff + f * D, D)]
        _accumulate(racc_v, rows_v, wf)
        return 0

    lax.fori_loop(0, F, fbody, 0)


def _reduce_out(racc_v, acc_v, out, b0):
    lanes = lax.iota(jnp.int32, 16)

    def red(i, _):
        rows16 = lanes + i * 16
        s = jnp.zeros((16,), jnp.float32)
        for d in range(D):
            col = jnp.full((16,), d, jnp.int32)
            s = s + plsc.load_gather(racc_v, [rows16, col])
        acc_v[pl.ds(i * 16, 16)] = s
        return 0

    lax.fori_loop(0, BPW // 16, red, 0)
    pltpu.sync_copy(acc_v, out.at[pl.ds(b0, BPW)])


def _sc_main_body(nf, f_base, ubs_t, tgt_f, item2d, wcat, out,
                  w_v, gidx_v, idx_v, rows_a, rows_b, racc_v, acc_v,
                  sem_a, sem_b):
    wid = lax.axis_index("s") * NC + lax.axis_index("c")
    b0 = wid * BPW

    pltpu.sync_copy(wcat, w_v)
    _zero_racc(racc_v)

    def fbody(f, _):
        pltpu.sync_copy(ubs_t.at[f, :, pl.ds(b0, BPW)], gidx_v)
        wf = w_v[pl.ds(F * D + (f_base + f) * D, D)]
        fv = f * V

        def bias_row(l, _):
            def add16(k, _):
                s = pl.ds(k * 16, 16)
                gidx_v[l, s] = gidx_v[l, s] + fv
                return 0

            lax.fori_loop(0, BPW // 16, add16, 0)
            return 0

        lax.fori_loop(0, L, bias_row, 0)

        pltpu.async_copy(item2d.at[gidx_v.at[0]], rows_a, sem_a)

        def lbody(l, _):
            @pl.when(l % 2 == 0)
            def _():
                @pl.when(l + 1 < L)
                def _():
                    pltpu.async_copy(item2d.at[gidx_v.at[l + 1]], rows_b,
                                     sem_b)
                pltpu.make_async_copy(item2d.at[gidx_v.at[l]], rows_a,
                                      sem_a).wait()
                _accumulate(racc_v, rows_a, wf)

            @pl.when(l % 2 == 1)
            def _():
                @pl.when(l + 1 < L)
                def _():
                    pltpu.async_copy(item2d.at[gidx_v.at[l + 1]], rows_a,
                                     sem_a)
                pltpu.make_async_copy(item2d.at[gidx_v.at[l]], rows_b,
                                      sem_b).wait()
                _accumulate(racc_v, rows_b, wf)

            return 0

        lax.fori_loop(0, L, lbody, 0)
        return 0

    lax.fori_loop(0, nf, fbody, 0)

    def tbody(f, _):
        pltpu.sync_copy(tgt_f.at[pl.ds((f_base + f) * B + b0, BPW)], idx_v)

        def add16(k, _):
            s = pl.ds(k * 16, 16)
            idx_v[s] = idx_v[s] + f * V
            return 0

        lax.fori_loop(0, BPW // 16, add16, 0)
        pltpu.async_copy(item2d.at[idx_v], rows_a, sem_a).wait()
        wf = w_v[pl.ds((f_base + f) * D, D)]
        _accumulate(racc_v, rows_a, wf)
        return 0

    lax.fori_loop(0, nf, tbody, 0)

    _reduce_out(racc_v, acc_v, out, b0)


def _sc_prof_body(prof_f, prof2d, wcat, out,
                  w_v, idx_v, rows_a, racc_v, acc_v, sem_a):
    wid = lax.axis_index("s") * NC + lax.axis_index("c")
    b0 = wid * BPW

    pltpu.sync_copy(wcat, w_v)
    _zero_racc(racc_v)
    _flat_gather_pass(prof_f, prof2d, 2 * F * D, b0, w_v, idx_v, rows_a,
                      racc_v, sem_a)
    _reduce_out(racc_v, acc_v, out, b0)


_SC_PARAMS = dict(
    out_type=jax.ShapeDtypeStruct((B,), jnp.float32),
    compiler_params=pltpu.CompilerParams(needs_layout_passes=False,
                                         use_tc_tiling_on_sc=False),
)


@functools.cache
def _sc_fns():
    mesh = plsc.VectorSubcoreMesh(core_axis_name="c", subcore_axis_name="s",
                                  num_cores=NC, num_subcores=NS)

    def make_main(nf, f_base):
        return functools.partial(
            pl.kernel,
            mesh=mesh,
            scratch_types=[
                pltpu.VMEM((3 * F * D,), jnp.float32),
                pltpu.VMEM((L, BPW), jnp.int32),
                pltpu.VMEM((BPW,), jnp.int32),
                pltpu.VMEM((BPW, D), jnp.float32),
                pltpu.VMEM((BPW, D), jnp.float32),
                pltpu.VMEM((BPW, D), jnp.float32),
                pltpu.VMEM((BPW,), jnp.float32),
                pltpu.SemaphoreType.DMA,
                pltpu.SemaphoreType.DMA,
            ],
            **_SC_PARAMS,
        )(functools.partial(_sc_main_body, nf, f_base))

    main_a = make_main(FA, 0)
    main_b = make_main(F - FA, FA)
    prof = functools.partial(
        pl.kernel,
        mesh=mesh,
        scratch_types=[
            pltpu.VMEM((3 * F * D,), jnp.float32),
            pltpu.VMEM((BPW,), jnp.int32),
            pltpu.VMEM((BPW, D), jnp.float32),
            pltpu.VMEM((BPW, D), jnp.float32),
            pltpu.VMEM((BPW,), jnp.float32),
            pltpu.SemaphoreType.DMA,
        ],
        **_SC_PARAMS,
    )(_sc_prof_body)
    return main_a, main_b, prof


def _head_body(s1_ref, s2_ref, s3_ref, ctx_ref, wc_ref, b_ref, o_ref):
    c = jnp.dot(ctx_ref[...], wc_ref[...], preferred_element_type=jnp.float32)
    logit = (s1_ref[0] + s2_ref[0] + s3_ref[0])[:, None] + c + b_ref[0, 0]
    o_ref[...] = jax.nn.sigmoid(logit)


def _head(s1, s2, s3, context, wc, bias):
    return pl.pallas_call(
        _head_body,
        out_shape=jax.ShapeDtypeStruct((B, 1), jnp.float32),
    )(s1, s2, s3, context, wc, bias)


def kernel(target_ad, ubs_feature, profile_feature, context_feature,
           item_emb, profile_emb, W, b):
    wt = W[:F * D, 0]
    wu = W[F * D:2 * F * D, 0] / L
    wp = W[2 * F * D:3 * F * D, 0]
    wc = W[3 * F * D:, :]
    wcat = jnp.concatenate([wt, wu, wp])

    item2d_a = item_emb[:FA].reshape(FA * V, D)
    item2d_b = item_emb[FA:].reshape((F - FA) * V, D)
    prof2d = profile_emb.reshape(F * V, D)
    ubs_t = jnp.transpose(ubs_feature, (2, 1, 0))
    ubs_ta = ubs_t[:FA]
    ubs_tb = ubs_t[FA:]
    tgt_f = target_ad.T.reshape(F * B)
    prof_f = profile_feature.T.reshape(F * B)

    main_a, main_b, prof_fn = _sc_fns()
    s1 = main_a(ubs_ta, tgt_f, item2d_a, wcat)
    s2 = main_b(ubs_tb, tgt_f, item2d_b, wcat)
    s3 = prof_fn(prof_f, prof2d, wcat)

    return _head(s1.reshape(1, B), s2.reshape(1, B), s3.reshape(1, B),
                 context_feature, wc, b.reshape(1, 1))


# speedup vs baseline: 1.0050x; 1.0050x over previous
import functools

import jax
import jax.numpy as jnp
from jax import lax
from jax.experimental import pallas as pl
from jax.experimental.pallas import tpu as pltpu
from jax.experimental.pallas import tpu_sc as plsc

B = 4096
L = 50
F = 13
V = 100000
D = 16
C = 16

NC = 2
NS = 16
NW = NC * NS
BPW = B // NW


def _zero_racc(racc_v):
    def zero16(i, _):
        racc_v[i] = jnp.zeros((D,), jnp.float32)
        return 0

    lax.fori_loop(0, BPW, zero16, 0)


def _accumulate(racc_v, rows_v, wf):
    def acc8(i, _):
        for j in range(8):
            plsc.addupdate(racc_v.at[i * 8 + j], rows_v[i * 8 + j] * wf)
        return 0

    lax.fori_loop(0, BPW // 8, acc8, 0)


def _flat_gather_pass(idx_src, table, woff, b0, w_v, idx_v, rows_v, racc_v,
                      sem):
    def fbody(f, _):
        pltpu.sync_copy(idx_src.at[pl.ds(f * B + b0, BPW)], idx_v)

        def add16(k, _):
            s = pl.ds(k * 16, 16)
            idx_v[s] = idx_v[s] + f * V
            return 0

        lax.fori_loop(0, BPW // 16, add16, 0)
        pltpu.async_copy(table.at[idx_v], rows_v, sem).wait()
        wf = w_v[pl.ds(woff + f * D, D)]
        _accumulate(racc_v, rows_v, wf)
        return 0

    lax.fori_loop(0, F, fbody, 0)


def _reduce_out(racc_v, acc_v, out, b0):
    lanes = lax.iota(jnp.int32, 16)

    def red(i, _):
        rows16 = lanes + i * 16
        s = jnp.zeros((16,), jnp.float32)
        for d in range(D):
            col = jnp.full((16,), d, jnp.int32)
            s = s + plsc.load_gather(racc_v, [rows16, col])
        acc_v[pl.ds(i * 16, 16)] = s
        return 0

    lax.fori_loop(0, BPW // 16, red, 0)
    pltpu.sync_copy(acc_v, out.at[pl.ds(b0, BPW)])


def _sc_main_body(ubs_t, tgt_f, item2d, wcat, out,
                  w_v, gidx_v, idx_v, rows_a, rows_b, racc_v, acc_v,
                  sem_a, sem_b):
    wid = lax.axis_index("s") * NC + lax.axis_index("c")
    b0 = wid * BPW

    pltpu.sync_copy(wcat, w_v)
    _zero_racc(racc_v)

    def fbody(f, _):
        pltpu.sync_copy(ubs_t.at[f, :, pl.ds(b0, BPW)], gidx_v)
        wf = w_v[pl.ds(F * D + f * D, D)]
        fv = f * V

        def bias_row(l, _):
            def add16(k, _):
                s = pl.ds(k * 16, 16)
                gidx_v[l, s] = gidx_v[l, s] + fv
                return 0

            lax.fori_loop(0, BPW // 16, add16, 0)
            return 0

        lax.fori_loop(0, L, bias_row, 0)

        pltpu.async_copy(item2d.at[gidx_v.at[0]], rows_a, sem_a)

        def lbody(l, _):
            @pl.when(l % 2 == 0)
            def _():
                @pl.when(l + 1 < L)
                def _():
                    pltpu.async_copy(item2d.at[gidx_v.at[l + 1]], rows_b,
                                     sem_b)
                pltpu.make_async_copy(item2d.at[gidx_v.at[l]], rows_a,
                                      sem_a).wait()
                _accumulate(racc_v, rows_a, wf)

            @pl.when(l % 2 == 1)
            def _():
                @pl.when(l + 1 < L)
                def _():
                    pltpu.async_copy(item2d.at[gidx_v.at[l + 1]], rows_a,
                                     sem_a)
                pltpu.make_async_copy(item2d.at[gidx_v.at[l]], rows_b,
                                      sem_b).wait()
                _accumulate(racc_v, rows_b, wf)

            return 0

        lax.fori_loop(0, L, lbody, 0)
        return 0

    lax.fori_loop(0, F, fbody, 0)

    _flat_gather_pass(tgt_f, item2d, 0, b0, w_v, idx_v, rows_a, racc_v,
                      sem_a)

    _reduce_out(racc_v, acc_v, out, b0)


def _sc_prof_body(prof_f, prof2d, wcat, out,
                  w_v, idx_v, rows_a, racc_v, acc_v, sem_a):
    wid = lax.axis_index("s") * NC + lax.axis_index("c")
    b0 = wid * BPW

    pltpu.sync_copy(wcat, w_v)
    _zero_racc(racc_v)
    _flat_gather_pass(prof_f, prof2d, 2 * F * D, b0, w_v, idx_v, rows_a,
                      racc_v, sem_a)
    _reduce_out(racc_v, acc_v, out, b0)


_SC_PARAMS = dict(
    out_type=jax.ShapeDtypeStruct((B,), jnp.float32),
    compiler_params=pltpu.CompilerParams(needs_layout_passes=False,
                                         use_tc_tiling_on_sc=False),
)


@functools.cache
def _sc_fns():
    mesh = plsc.VectorSubcoreMesh(core_axis_name="c", subcore_axis_name="s",
                                  num_cores=NC, num_subcores=NS)
    main = functools.partial(
        pl.kernel,
        mesh=mesh,
        scratch_types=[
            pltpu.VMEM((3 * F * D,), jnp.float32),
            pltpu.VMEM((L, BPW), jnp.int32),
            pltpu.VMEM((BPW,), jnp.int32),
            pltpu.VMEM((BPW, D), jnp.float32),
            pltpu.VMEM((BPW, D), jnp.float32),
            pltpu.VMEM((BPW, D), jnp.float32),
            pltpu.VMEM((BPW,), jnp.float32),
            pltpu.SemaphoreType.DMA,
            pltpu.SemaphoreType.DMA,
        ],
        **_SC_PARAMS,
    )(_sc_main_body)
    prof = functools.partial(
        pl.kernel,
        mesh=mesh,
        scratch_types=[
            pltpu.VMEM((3 * F * D,), jnp.float32),
            pltpu.VMEM((BPW,), jnp.int32),
            pltpu.VMEM((BPW, D), jnp.float32),
            pltpu.VMEM((BPW, D), jnp.float32),
            pltpu.VMEM((BPW,), jnp.float32),
            pltpu.SemaphoreType.DMA,
        ],
        **_SC_PARAMS,
    )(_sc_prof_body)
    return main, prof


def _head_body(s1_ref, s2_ref, ctx_ref, wc_ref, b_ref, o_ref):
    c = jnp.dot(ctx_ref[...], wc_ref[...], preferred_element_type=jnp.float32)
    logit = (s1_ref[0] + s2_ref[0])[:, None] + c + b_ref[0, 0]
    o_ref[...] = jax.nn.sigmoid(logit)


def _head(s1, s2, context, wc, bias):
    return pl.pallas_call(
        _head_body,
        out_shape=jax.ShapeDtypeStruct((B, 1), jnp.float32),
    )(s1, s2, context, wc, bias)


def kernel(target_ad, ubs_feature, profile_feature, context_feature,
           item_emb, profile_emb, W, b):
    wt = W[:F * D, 0]
    wu = W[F * D:2 * F * D, 0] / L
    wp = W[2 * F * D:3 * F * D, 0]
    wc = W[3 * F * D:, :]
    wcat = jnp.concatenate([wt, wu, wp])

    item2d = item_emb.reshape(F * V, D)
    prof2d = profile_emb.reshape(F * V, D)
    ubs_t = jnp.transpose(ubs_feature, (2, 1, 0))
    tgt_f = target_ad.T.reshape(F * B)
    prof_f = profile_feature.T.reshape(F * B)

    main_fn, prof_fn = _sc_fns()
    s1 = main_fn(ubs_t, tgt_f, item2d, wcat)
    s2 = prof_fn(prof_f, prof2d, wcat)

    return _head(s1.reshape(1, B), s2.reshape(1, B), context_feature, wc,
                 b.reshape(1, 1))
